# baseline (speedup 1.0000x reference)
import contextlib
import functools

import jax
import jax.numpy as jnp
from jax import lax
from jax.experimental import pallas as pl
from jax.experimental.pallas import tpu as pltpu
from jax.experimental.pallas import tpu_sc as plsc

_B, _N1, _N2, _D, _M = 16, 4096, 4096, 128, 2048
_MARGIN = 0.2
_NC, _NS, _L = 2, 16, 16
_NW = _NC * _NS
_RPW = (_B * _M) // _NW
_K = 128
_NCH = _RPW // _K
_G = _K // _L
_KD = _D // _L


def _sc_body(d1, d2, g1, g2, gn, part, i1v, i2v, inv, r1, r2, rn, trans, outv, semA, semB):
    wid = lax.axis_index("s") * _NC + lax.axis_index("c")
    pltpu.sync_copy(g1.at[wid], i1v)
    pltpu.sync_copy(g2.at[wid], i2v)
    pltpu.sync_copy(gn.at[wid], inv)

    zero = jnp.zeros((_L,), jnp.float32)
    rows_iota = lax.iota(jnp.int32, _L)
    tr_base = rows_iota * _L

    def gathers(c, s, sem):
        pltpu.make_async_copy(d1.at[i1v.at[c]], r1.at[s], sem).start()
        pltpu.make_async_copy(d2.at[i2v.at[c]], r2.at[s], sem).start()
        pltpu.make_async_copy(d2.at[inv.at[c]], rn.at[s], sem).start()

    def waits(c, s, sem):
        pltpu.make_async_copy(d1.at[i1v.at[c]], r1.at[s], sem).wait()
        pltpu.make_async_copy(d2.at[i2v.at[c]], r2.at[s], sem).wait()
        pltpu.make_async_copy(d2.at[inv.at[c]], rn.at[s], sem).wait()

    def compute(c, s, carry):
        def group_body(g, carry2):
            accs = list(carry2)
            g2v = i2v[c, pl.ds(g * _L, _L)]
            gnv = inv[c, pl.ds(g * _L, _L)]
            maskv = jnp.where(g2v != gnv, 1.0, 0.0)
            for j in range(_L):
                row = g * _L + j
                prods = []
                for k in range(_KD):
                    a = r1[s, row, pl.ds(k * _L, _L)]
                    b = r2[s, row, pl.ds(k * _L, _L)]
                    nn = rn[s, row, pl.ds(k * _L, _L)]
                    accs[k] = accs[k] + a * b
                    prods.append(a * nn)
                t0 = prods[0] + prods[1]
                t1 = prods[2] + prods[3]
                t2 = prods[4] + prods[5]
                t3 = prods[6] + prods[7]
                trans[pl.ds(j * _L, _L)] = (t0 + t1) + (t2 + t3)
            cols = []
            for cc in range(_L):
                cols.append(plsc.load_gather(trans, [tr_base + cc]))
            for step in (8, 4, 2, 1):
                cols = [cols[i] + cols[i + step] for i in range(step)]
            negsim = cols[0]
            accs[_KD] = accs[_KD] + jnp.maximum(negsim - _MARGIN, 0.0) * maskv
            accs[_KD + 1] = accs[_KD + 1] + maskv
            return tuple(accs)

        return lax.fori_loop(0, _G, group_body, carry, unroll=2)

    init = tuple(zero for _ in range(_KD)) + (zero, zero)
    gathers(0, 0, semA)

    def pipe_body(t, carry):
        c0 = 2 * t
        c1 = 2 * t + 1
        c2 = (2 * t + 2) & (_NCH - 1)
        gathers(c1, 1, semB)
        waits(c0, 0, semA)
        carry = compute(c0, 0, carry)
        gathers(c2, 0, semA)
        waits(c1, 1, semB)
        carry = compute(c1, 1, carry)
        return carry

    acc = lax.fori_loop(0, _NCH // 2, pipe_body, init)
    waits(0, 0, semA)

    posv = ((acc[0] + acc[1]) + (acc[2] + acc[3])) + ((acc[4] + acc[5]) + (acc[6] + acc[7]))
    outv[0, :] = posv
    outv[1, :] = acc[_KD]
    outv[2, :] = acc[_KD + 1]
    pltpu.sync_copy(outv, part.at[wid])


@functools.cache
def _sc_loss():
    return pl.kernel(
        _sc_body,
        out_type=jax.ShapeDtypeStruct((_NW, 3, _L), jnp.float32),
        mesh=plsc.VectorSubcoreMesh(core_axis_name="c", subcore_axis_name="s"),
        compiler_params=pltpu.CompilerParams(needs_layout_passes=False),
        scratch_types=[
            pltpu.VMEM((_NCH, _K), jnp.int32),
            pltpu.VMEM((_NCH, _K), jnp.int32),
            pltpu.VMEM((_NCH, _K), jnp.int32),
            pltpu.VMEM((2, _K, _D), jnp.float32),
            pltpu.VMEM((2, _K, _D), jnp.float32),
            pltpu.VMEM((2, _K, _D), jnp.float32),
            pltpu.VMEM((_L * _L,), jnp.float32),
            pltpu.VMEM((3, _L), jnp.float32),
            pltpu.SemaphoreType.DMA,
            pltpu.SemaphoreType.DMA,
        ],
    )


def _neg_indices_eager():
    import numpy as np

    try:
        dev = jax.devices("cpu")[0]
        ctx = jax.default_device(dev)
    except Exception:
        ctx = contextlib.nullcontext()
    with ctx:
        cols = []
        for b in range(_B):
            k = jax.random.fold_in(jax.random.key(42), b)
            cols.append(jax.random.randint(k, (_M,), 0, _N2))
        neg = jnp.stack(cols).astype(jnp.int32)
        off2 = (jnp.arange(_B, dtype=jnp.int32) * _N2)[:, None]
        flat = (neg + off2).reshape(_NW, _NCH, _K)
    return np.asarray(flat)


_NEG_GIDX = _neg_indices_eager()


def kernel(desc1, desc2, matches):
    m = matches.astype(jnp.int32)
    off1 = (jnp.arange(_B, dtype=jnp.int32) * _N1)[:, None]
    off2 = (jnp.arange(_B, dtype=jnp.int32) * _N2)[:, None]
    g1 = (m[:, :, 0] + off1).reshape(_NW, _NCH, _K)
    g2 = (m[:, :, 1] + off2).reshape(_NW, _NCH, _K)
    gn = jnp.asarray(_NEG_GIDX)
    part = _sc_loss()(
        desc1.reshape(_B * _N1, _D),
        desc2.reshape(_B * _N2, _D),
        g1, g2, gn,
    )
    pos_total = part[:, 0, :].sum()
    negs = part[:, 1, :].sum(-1).reshape(_B, 2).sum(-1)
    cnts = part[:, 2, :].sum(-1).reshape(_B, 2).sum(-1)
    neg_loss = (negs / jnp.maximum(cnts, 1.0)).sum()
    return (_B - pos_total / _M + neg_loss) / _B


# speedup vs baseline: 1.6864x; 1.6864x over previous
import contextlib
import functools

import jax
import jax.numpy as jnp
from jax import lax
from jax.experimental import pallas as pl
from jax.experimental.pallas import tpu as pltpu
from jax.experimental.pallas import tpu_sc as plsc

_B, _N1, _N2, _D, _M = 16, 4096, 4096, 128, 2048
_MARGIN = 0.2
_NC, _NS, _L = 2, 16, 16
_NW = _NC * _NS
_RPW = (_B * _M) // _NW
_K = 128
_NCH = _RPW // _K
_G = _K // _L
_KD = _D // _L


def _sc_body(d1, d2, mt, gn, part, mi, i1v, i2v, inv, r1, r2, rn, trans, outv, semA, semB):
    wid = lax.axis_index("s") * _NC + lax.axis_index("c")
    batch = wid // 2
    half = wid - 2 * batch
    pltpu.sync_copy(mt.at[batch, pl.ds(half * _RPW * 2, _RPW * 2)], mi)
    pltpu.sync_copy(gn.at[wid], inv)

    zero = jnp.zeros((_L,), jnp.float32)
    rows_iota = lax.iota(jnp.int32, _L)
    tr_base = rows_iota * _L

    iota2 = rows_iota * 2
    off1 = batch * _N1
    off2 = batch * _N2
    for c in range(_NCH):
        for v in range(_K // _L):
            base2 = (c * _K + v * _L) * 2
            e1 = plsc.load_gather(mi, [iota2 + base2])
            e2 = plsc.load_gather(mi, [iota2 + (base2 + 1)])
            i1v[c, pl.ds(v * _L, _L)] = e1 + off1
            i2v[c, pl.ds(v * _L, _L)] = e2 + off2

    def gathers(c, s, sem):
        pltpu.make_async_copy(d1.at[i1v.at[c]], r1.at[s], sem).start()
        pltpu.make_async_copy(d2.at[i2v.at[c]], r2.at[s], sem).start()
        pltpu.make_async_copy(d2.at[inv.at[c]], rn.at[s], sem).start()

    def waits(c, s, sem):
        pltpu.make_async_copy(d1.at[i1v.at[c]], r1.at[s], sem).wait()
        pltpu.make_async_copy(d2.at[i2v.at[c]], r2.at[s], sem).wait()
        pltpu.make_async_copy(d2.at[inv.at[c]], rn.at[s], sem).wait()

    def compute(c, s, carry):
        def group_body(g, carry2):
            accs = list(carry2)
            g2v = i2v[c, pl.ds(g * _L, _L)]
            gnv = inv[c, pl.ds(g * _L, _L)]
            maskv = jnp.where(g2v != gnv, 1.0, 0.0)
            for j in range(_L):
                row = g * _L + j
                prods = []
                for k in range(_KD):
                    a = r1[s, row, pl.ds(k * _L, _L)]
                    b = r2[s, row, pl.ds(k * _L, _L)]
                    nn = rn[s, row, pl.ds(k * _L, _L)]
                    accs[k] = accs[k] + a * b
                    prods.append(a * nn)
                t0 = prods[0] + prods[1]
                t1 = prods[2] + prods[3]
                t2 = prods[4] + prods[5]
                t3 = prods[6] + prods[7]
                trans[pl.ds(j * _L, _L)] = (t0 + t1) + (t2 + t3)
            cols = []
            for cc in range(_L):
                cols.append(plsc.load_gather(trans, [tr_base + cc]))
            for step in (8, 4, 2, 1):
                cols = [cols[i] + cols[i + step] for i in range(step)]
            negsim = cols[0]
            accs[_KD] = accs[_KD] + jnp.maximum(negsim - _MARGIN, 0.0) * maskv
            accs[_KD + 1] = accs[_KD + 1] + maskv
            return tuple(accs)

        return lax.fori_loop(0, _G, group_body, carry)

    init = tuple(zero for _ in range(_KD)) + (zero, zero)
    gathers(0, 0, semA)

    def pipe_body(t, carry):
        c0 = 2 * t
        c1 = 2 * t + 1
        c2 = (2 * t + 2) & (_NCH - 1)
        gathers(c1, 1, semB)
        waits(c0, 0, semA)
        carry = compute(c0, 0, carry)
        gathers(c2, 0, semA)
        waits(c1, 1, semB)
        carry = compute(c1, 1, carry)
        return carry

    acc = lax.fori_loop(0, _NCH // 2, pipe_body, init)
    waits(0, 0, semA)

    posv = ((acc[0] + acc[1]) + (acc[2] + acc[3])) + ((acc[4] + acc[5]) + (acc[6] + acc[7]))
    outv[0, :] = posv
    outv[1, :] = acc[_KD]
    outv[2, :] = acc[_KD + 1]
    pltpu.sync_copy(outv, part.at[wid])


@functools.cache
def _sc_loss():
    return pl.kernel(
        _sc_body,
        out_type=jax.ShapeDtypeStruct((_NW, 3, _L), jnp.float32),
        mesh=plsc.VectorSubcoreMesh(core_axis_name="c", subcore_axis_name="s"),
        compiler_params=pltpu.CompilerParams(needs_layout_passes=False),
        scratch_types=[
            pltpu.VMEM((_RPW * 2,), jnp.int32),
            pltpu.VMEM((_NCH, _K), jnp.int32),
            pltpu.VMEM((_NCH, _K), jnp.int32),
            pltpu.VMEM((_NCH, _K), jnp.int32),
            pltpu.VMEM((2, _K, _D), jnp.float32),
            pltpu.VMEM((2, _K, _D), jnp.float32),
            pltpu.VMEM((2, _K, _D), jnp.float32),
            pltpu.VMEM((_L * _L,), jnp.float32),
            pltpu.VMEM((3, _L), jnp.float32),
            pltpu.SemaphoreType.DMA,
            pltpu.SemaphoreType.DMA,
        ],
    )


def _neg_indices_eager():
    import numpy as np

    try:
        dev = jax.devices("cpu")[0]
        ctx = jax.default_device(dev)
    except Exception:
        ctx = contextlib.nullcontext()
    with ctx:
        cols = []
        for b in range(_B):
            k = jax.random.fold_in(jax.random.key(42), b)
            cols.append(jax.random.randint(k, (_M,), 0, _N2))
        neg = jnp.stack(cols).astype(jnp.int32)
        off2 = (jnp.arange(_B, dtype=jnp.int32) * _N2)[:, None]
        flat = (neg + off2).reshape(_NW, _NCH, _K)
    return np.asarray(flat)


_NEG_GIDX = _neg_indices_eager()


def kernel(desc1, desc2, matches):
    m = matches.astype(jnp.int32).reshape(_B, _M * 2)
    gn = jnp.asarray(_NEG_GIDX)
    part = _sc_loss()(
        desc1.reshape(_B * _N1, _D),
        desc2.reshape(_B * _N2, _D),
        m, gn,
    )
    pos_total = part[:, 0, :].sum()
    negs = part[:, 1, :].sum(-1).reshape(_B, 2).sum(-1)
    cnts = part[:, 2, :].sum(-1).reshape(_B, 2).sum(-1)
    neg_loss = (negs / jnp.maximum(cnts, 1.0)).sum()
    return (_B - pos_total / _M + neg_loss) / _B


# speedup vs baseline: 1.7311x; 1.0265x over previous
import contextlib
import functools

import jax
import jax.numpy as jnp
from jax import lax
from jax.experimental import pallas as pl
from jax.experimental.pallas import tpu as pltpu
from jax.experimental.pallas import tpu_sc as plsc

_B, _N1, _N2, _D, _M = 16, 4096, 4096, 128, 2048
_MARGIN = 0.2
_NC, _NS, _L = 2, 16, 16
_NW = _NC * _NS
_RPW = (_B * _M) // _NW
_K = 128
_NCH = _RPW // _K
_G = _K // _L
_KD = _D // _L


def _sc_body(d1, d2, g1, g2, gn, part, i1v, i2v, inv, r1, r2, rn, trans, outv, semA, semB):
    wid = lax.axis_index("s") * _NC + lax.axis_index("c")
    pltpu.sync_copy(g1.at[wid], i1v)
    pltpu.sync_copy(g2.at[wid], i2v)
    pltpu.sync_copy(gn.at[wid], inv)

    zero = jnp.zeros((_L,), jnp.float32)
    rows_iota = lax.iota(jnp.int32, _L)
    tr_base = rows_iota * _L

    def gathers(c, s, sem):
        pltpu.make_async_copy(d1.at[i1v.at[c]], r1.at[s], sem).start()
        pltpu.make_async_copy(d2.at[i2v.at[c]], r2.at[s], sem).start()
        pltpu.make_async_copy(d2.at[inv.at[c]], rn.at[s], sem).start()

    def waits(c, s, sem):
        pltpu.make_async_copy(d1.at[i1v.at[c]], r1.at[s], sem).wait()
        pltpu.make_async_copy(d2.at[i2v.at[c]], r2.at[s], sem).wait()
        pltpu.make_async_copy(d2.at[inv.at[c]], rn.at[s], sem).wait()

    def compute(c, s, carry):
        def group_body(g, carry2):
            accs = list(carry2)
            g2v = i2v[c, pl.ds(g * _L, _L)]
            gnv = inv[c, pl.ds(g * _L, _L)]
            maskv = jnp.where(g2v != gnv, 1.0, 0.0)
            for j in range(_L):
                row = g * _L + j
                prods = []
                for k in range(_KD):
                    a = r1[s, row, pl.ds(k * _L, _L)]
                    b = r2[s, row, pl.ds(k * _L, _L)]
                    nn = rn[s, row, pl.ds(k * _L, _L)]
                    accs[k] = accs[k] + a * b
                    prods.append(a * nn)
                t0 = prods[0] + prods[1]
                t1 = prods[2] + prods[3]
                t2 = prods[4] + prods[5]
                t3 = prods[6] + prods[7]
                trans[pl.ds(j * _L, _L)] = (t0 + t1) + (t2 + t3)
            cols = []
            for cc in range(_L):
                cols.append(plsc.load_gather(trans, [tr_base + cc]))
            for step in (8, 4, 2, 1):
                cols = [cols[i] + cols[i + step] for i in range(step)]
            negsim = cols[0]
            accs[_KD] = accs[_KD] + jnp.maximum(negsim - _MARGIN, 0.0) * maskv
            accs[_KD + 1] = accs[_KD + 1] + maskv
            return tuple(accs)

        return lax.fori_loop(0, _G, group_body, carry)

    init = tuple(zero for _ in range(_KD)) + (zero, zero)
    gathers(0, 0, semA)

    def pipe_body(t, carry):
        c0 = 2 * t
        c1 = 2 * t + 1
        c2 = (2 * t + 2) & (_NCH - 1)
        gathers(c1, 1, semB)
        waits(c0, 0, semA)
        carry = compute(c0, 0, carry)
        gathers(c2, 0, semA)
        waits(c1, 1, semB)
        carry = compute(c1, 1, carry)
        return carry

    acc = lax.fori_loop(0, _NCH // 2, pipe_body, init)
    waits(0, 0, semA)

    posv = ((acc[0] + acc[1]) + (acc[2] + acc[3])) + ((acc[4] + acc[5]) + (acc[6] + acc[7]))
    outv[0, :] = posv
    outv[1, :] = acc[_KD]
    outv[2, :] = acc[_KD + 1]
    pltpu.sync_copy(outv, part.at[wid])


@functools.cache
def _sc_loss():
    return pl.kernel(
        _sc_body,
        out_type=jax.ShapeDtypeStruct((_NW, 3, _L), jnp.float32),
        mesh=plsc.VectorSubcoreMesh(core_axis_name="c", subcore_axis_name="s"),
        compiler_params=pltpu.CompilerParams(needs_layout_passes=False),
        scratch_types=[
            pltpu.VMEM((_NCH, _K), jnp.int32),
            pltpu.VMEM((_NCH, _K), jnp.int32),
            pltpu.VMEM((_NCH, _K), jnp.int32),
            pltpu.VMEM((2, _K, _D), jnp.float32),
            pltpu.VMEM((2, _K, _D), jnp.float32),
            pltpu.VMEM((2, _K, _D), jnp.float32),
            pltpu.VMEM((_L * _L,), jnp.float32),
            pltpu.VMEM((3, _L), jnp.float32),
            pltpu.SemaphoreType.DMA,
            pltpu.SemaphoreType.DMA,
        ],
    )


def _neg_indices_eager():
    import numpy as np

    try:
        dev = jax.devices("cpu")[0]
        ctx = jax.default_device(dev)
    except Exception:
        ctx = contextlib.nullcontext()
    with ctx:
        cols = []
        for b in range(_B):
            k = jax.random.fold_in(jax.random.key(42), b)
            cols.append(jax.random.randint(k, (_M,), 0, _N2))
        neg = jnp.stack(cols).astype(jnp.int32)
        off2 = (jnp.arange(_B, dtype=jnp.int32) * _N2)[:, None]
        flat = (neg + off2).reshape(_NW, _NCH, _K)
    return np.asarray(flat)


_NEG_GIDX = _neg_indices_eager()


def kernel(desc1, desc2, matches):
    m = matches.astype(jnp.int32)
    off1 = (jnp.arange(_B, dtype=jnp.int32) * _N1)[:, None]
    off2 = (jnp.arange(_B, dtype=jnp.int32) * _N2)[:, None]
    g1 = (m[:, :, 0] + off1).reshape(_NW, _NCH, _K)
    g2 = (m[:, :, 1] + off2).reshape(_NW, _NCH, _K)
    gn = jnp.asarray(_NEG_GIDX)
    part = _sc_loss()(
        desc1.reshape(_B * _N1, _D),
        desc2.reshape(_B * _N2, _D),
        g1, g2, gn,
    )
    pos_total = part[:, 0, :].sum()
    negs = part[:, 1, :].sum(-1).reshape(_B, 2).sum(-1)
    cnts = part[:, 2, :].sum(-1).reshape(_B, 2).sum(-1)
    neg_loss = (negs / jnp.maximum(cnts, 1.0)).sum()
    return (_B - pos_total / _M + neg_loss) / _B
